# Initial kernel scaffold; baseline (speedup 1.0000x reference)
import functools

import jax
import jax.numpy as jnp
from jax import lax
from jax.experimental import pallas as pl
from jax.experimental.pallas import tpu as pltpu
from jax.experimental.pallas import tpu_sc as plsc

_NC = 2
_NS = 16
_NW = _NC * _NS
_SB = 80
_L = 16


def _sc_mesh():
    return plsc.VectorSubcoreMesh(core_axis_name="c", subcore_axis_name="s")


def _deg_body(n, nb, dst3, out, dst2d, ones_v, stage, acc):
    c = lax.axis_index("c")
    s = lax.axis_index("s")
    w = c * _NS + s
    stripe = n // _NS
    nchunk = stripe // 125

    zeros = jnp.zeros((_L,), jnp.float32)
    ones = jnp.ones((_L,), jnp.float32)
    for i in range(125):
        stage[i, :] = zeros
    for i in range(_SB):
        ones_v[i, :] = ones
    for k in range(nchunk):
        pltpu.sync_copy(stage, acc.at[pl.ds(s * stripe + k * 125, 125)])
    plsc.subcore_barrier()

    pltpu.sync_copy(dst3.at[w], dst2d)

    def body(j, carry):
        pltpu.sync_copy(ones_v, acc.at[dst2d.at[j]], add=True)
        return carry

    lax.fori_loop(0, nb, body, 0)
    plsc.subcore_barrier()

    for k in range(nchunk):
        r0 = s * stripe + k * 125
        pltpu.sync_copy(acc.at[pl.ds(r0, 125)], stage)
        pltpu.sync_copy(stage, out.at[c, pl.ds(r0, 125)])


def _sc_degree(dst3, n, nb):
    body = functools.partial(_deg_body, n, nb)
    k = pl.kernel(
        body,
        out_type=jax.ShapeDtypeStruct((_NC, n, _L), jnp.float32),
        mesh=_sc_mesh(),
        scratch_types=[
            pltpu.VMEM((nb, _SB), jnp.int32),
            pltpu.VMEM((_SB, _L), jnp.float32),
            pltpu.VMEM((125, _L), jnp.float32),
            pltpu.VMEM_SHARED((n, _L), jnp.float32),
        ],
    )
    return k(dst3)


def _agg_body(n, nb, table, src3, dst3, out, src2d, dst2d, rows, stage, acc,
              sem):
    c = lax.axis_index("c")
    s = lax.axis_index("s")
    w = c * _NS + s
    stripe = n // _NS
    nchunk = stripe // 125

    zeros = jnp.zeros((_L,), jnp.float32)
    for i in range(125):
        for cc in range(4):
            stage[i, pl.ds(cc * _L, _L)] = zeros
    for k in range(nchunk):
        pltpu.sync_copy(stage, acc.at[pl.ds(s * stripe + k * 125, 125)])
    plsc.subcore_barrier()

    pltpu.sync_copy(src3.at[w], src2d)
    pltpu.sync_copy(dst3.at[w], dst2d)

    def body(j, carry):
        pltpu.async_copy(table.at[src2d.at[j]], rows, sem).wait()
        pltpu.sync_copy(rows, acc.at[dst2d.at[j]], add=True)
        return carry

    lax.fori_loop(0, nb, body, 0)
    plsc.subcore_barrier()

    for k in range(nchunk):
        r0 = s * stripe + k * 125
        pltpu.sync_copy(acc.at[pl.ds(r0, 125)], stage)
        pltpu.sync_copy(stage, out.at[c, pl.ds(r0, 125)])


def _sc_aggregate(table, src3, dst3, n, nb):
    body = functools.partial(_agg_body, n, nb)
    k = pl.kernel(
        body,
        out_type=jax.ShapeDtypeStruct((_NC, n, 64), jnp.float32),
        mesh=_sc_mesh(),
        scratch_types=[
            pltpu.VMEM((nb, _SB), jnp.int32),
            pltpu.VMEM((nb, _SB), jnp.int32),
            pltpu.VMEM((_SB, 64), jnp.float32),
            pltpu.VMEM((125, 64), jnp.float32),
            pltpu.VMEM_SHARED((n, 64), jnp.float32),
            pltpu.SemaphoreType.DMA,
        ],
    )
    return k(table, src3, dst3)


def _mm_body(x_ref, w_ref, o_ref):
    o_ref[...] = jnp.dot(x_ref[...], w_ref[...],
                         preferred_element_type=jnp.float32)


def _scale_body(degp_ref, h1_ref, dis_ref, hs1_ref):
    degp = degp_ref[...]
    deg = degp[0, :, 0] + degp[1, :, 0] + 1.0
    dis = lax.rsqrt(deg)
    dis_ref[...] = dis
    hs1_ref[...] = h1_ref[...] * dis[:, None]


def _hidden_body(rawp_ref, h1_ref, dis_ref, b1_ref, h_ref, hs2_ref):
    rawp = rawp_ref[...]
    raw = rawp[0] + rawp[1]
    dis = dis_ref[...]
    a1 = dis[:, None] * raw + (dis * dis)[:, None] * h1_ref[...] \
        + b1_ref[...][None, :]
    h = jnp.maximum(a1, 0.0)
    h_ref[...] = h
    hs2_ref[...] = h * dis[:, None]


def _head_body(rawp_ref, h_ref, dis_ref, wmu_ref, bmu_ref, wvar_ref,
               bvar_ref, eps_ref, zm_ref, zv_ref, z_ref):
    rawp = rawp_ref[...]
    raw = rawp[0] + rawp[1]
    dis = dis_ref[...]
    a2 = dis[:, None] * raw + (dis * dis)[:, None] * h_ref[...]
    zm = jnp.dot(a2, wmu_ref[...], preferred_element_type=jnp.float32) \
        + bmu_ref[...][None, :]
    pv = jnp.dot(a2, wvar_ref[...], preferred_element_type=jnp.float32) \
        + bvar_ref[...][None, :]
    zv = jnp.maximum(pv, 0.0) + jnp.log(1.0 + jnp.exp(-jnp.abs(pv)))
    zm_ref[...] = zm
    zv_ref[...] = zv
    z_ref[...] = zm + zv * eps_ref[...]


def kernel(x, edge_index, W1, b1, Wmu, bmu, Wvar, bvar):
    n, d = x.shape
    e = edge_index.shape[1]
    h = W1.shape[1]
    z = Wmu.shape[1]
    nb = e // (_NW * _SB)

    src3 = edge_index[0].reshape(_NW, nb, _SB)
    dst3 = edge_index[1].reshape(_NW, nb, _SB)

    f32 = jnp.float32
    h1 = pl.pallas_call(
        _mm_body, out_shape=jax.ShapeDtypeStruct((n, h), f32))(x, W1)

    degp = _sc_degree(dst3, n, nb)

    dis, hs1 = pl.pallas_call(
        _scale_body,
        out_shape=(jax.ShapeDtypeStruct((n,), f32),
                   jax.ShapeDtypeStruct((n, h), f32)))(degp, h1)

    raw1p = _sc_aggregate(hs1, src3, dst3, n, nb)

    hh, hs2 = pl.pallas_call(
        _hidden_body,
        out_shape=(jax.ShapeDtypeStruct((n, h), f32),
                   jax.ShapeDtypeStruct((n, h), f32)))(raw1p, h1, dis, b1)

    raw2p = _sc_aggregate(hs2, src3, dst3, n, nb)

    eps = jax.random.normal(jax.random.key(42), (n, z), f32)
    zm, zv, zz = pl.pallas_call(
        _head_body,
        out_shape=(jax.ShapeDtypeStruct((n, z), f32),
                   jax.ShapeDtypeStruct((n, z), f32),
                   jax.ShapeDtypeStruct((n, z), f32)))(
        raw2p, hh, dis, Wmu, bmu, Wvar, bvar, eps)
    return (zm, zv, zz)


# speedup vs baseline: 27.3927x; 27.3927x over previous
import functools

import jax
import jax.numpy as jnp
from jax import lax
from jax.experimental import pallas as pl
from jax.experimental.pallas import tpu as pltpu
from jax.experimental.pallas import tpu_sc as plsc

_NC = 2
_NS = 16
_NW = _NC * _NS
_SB = 80
_L = 16
_NP = 10240


def _sc_mesh():
    return plsc.VectorSubcoreMesh(core_axis_name="c", subcore_axis_name="s")


def _sc_params():
    return pltpu.CompilerParams(use_tc_tiling_on_sc=False)


def _deg_body(nb, dst3, out, dst2d, ones_v, stage, acc):
    c = lax.axis_index("c")
    s = lax.axis_index("s")
    w = c * _NS + s
    stripe = _NP // _NS
    nchunk = stripe // 128

    zeros = jnp.zeros((_L,), jnp.float32)
    ones = jnp.ones((_L,), jnp.float32)
    for i in range(128):
        stage[i, :] = zeros
    for i in range(_SB):
        ones_v[i, :] = ones
    for k in range(nchunk):
        pltpu.sync_copy(stage, acc.at[pl.ds(s * stripe + k * 128, 128)])
    plsc.subcore_barrier()

    pltpu.sync_copy(dst3.at[w], dst2d)

    def body(j, carry):
        pltpu.sync_copy(ones_v, acc.at[dst2d.at[j]], add=True)
        return carry

    lax.fori_loop(0, nb, body, 0)
    plsc.subcore_barrier()

    for k in range(nchunk):
        r0 = s * stripe + k * 128
        pltpu.sync_copy(acc.at[pl.ds(r0, 128)], stage)
        pltpu.sync_copy(stage, out.at[c, pl.ds(r0, 128)])


def _sc_degree(dst3, nb):
    body = functools.partial(_deg_body, nb)
    k = pl.kernel(
        body,
        out_type=jax.ShapeDtypeStruct((_NC, _NP, _L), jnp.float32),
        mesh=_sc_mesh(),
        compiler_params=_sc_params(),
        scratch_types=[
            pltpu.VMEM((nb, _SB), jnp.int32),
            pltpu.VMEM((_SB, _L), jnp.float32),
            pltpu.VMEM((128, _L), jnp.float32),
            pltpu.VMEM_SHARED((_NP, _L), jnp.float32),
        ],
    )
    return k(dst3)


def _agg_body(nb, table, src3, dst3, out, src2d, dst2d, rows, stage, acc,
              sem):
    c = lax.axis_index("c")
    s = lax.axis_index("s")
    w = c * _NS + s
    stripe = _NP // _NS
    nchunk = stripe // 128

    zeros = jnp.zeros((_L,), jnp.float32)
    for i in range(128):
        for cc in range(4):
            stage[i, pl.ds(cc * _L, _L)] = zeros
    for k in range(nchunk):
        pltpu.sync_copy(stage, acc.at[pl.ds(s * stripe + k * 128, 128)])
    plsc.subcore_barrier()

    pltpu.sync_copy(src3.at[w], src2d)
    pltpu.sync_copy(dst3.at[w], dst2d)

    def body(j, carry):
        pltpu.async_copy(table.at[src2d.at[j]], rows, sem).wait()
        pltpu.sync_copy(rows, acc.at[dst2d.at[j]], add=True)
        return carry

    lax.fori_loop(0, nb, body, 0)
    plsc.subcore_barrier()

    for k in range(nchunk):
        r0 = s * stripe + k * 128
        pltpu.sync_copy(acc.at[pl.ds(r0, 128)], stage)
        pltpu.sync_copy(stage, out.at[c, pl.ds(r0, 128)])


def _sc_aggregate(table, src3, dst3, nb):
    body = functools.partial(_agg_body, nb)
    k = pl.kernel(
        body,
        out_type=jax.ShapeDtypeStruct((_NC, _NP, 64), jnp.float32),
        mesh=_sc_mesh(),
        compiler_params=_sc_params(),
        scratch_types=[
            pltpu.VMEM((nb, _SB), jnp.int32),
            pltpu.VMEM((nb, _SB), jnp.int32),
            pltpu.VMEM((_SB, 64), jnp.float32),
            pltpu.VMEM((128, 64), jnp.float32),
            pltpu.VMEM_SHARED((_NP, 64), jnp.float32),
            pltpu.SemaphoreType.DMA,
        ],
    )
    return k(table, src3, dst3)


def _mm_body(x_ref, w_ref, o_ref):
    o_ref[...] = jnp.dot(x_ref[...], w_ref[...],
                         preferred_element_type=jnp.float32)


def _scale_body(degp_ref, h1_ref, dis_ref, hs1_ref):
    n = h1_ref.shape[0]
    degp = degp_ref[...]
    deg = degp[0, :n, 0] + degp[1, :n, 0] + 1.0
    dis = lax.rsqrt(deg)
    dis_ref[...] = dis
    hs1_ref[...] = h1_ref[...] * dis[:, None]


def _hidden_body(rawp_ref, h1_ref, dis_ref, b1_ref, h_ref, hs2_ref):
    n = h1_ref.shape[0]
    rawp = rawp_ref[...]
    raw = rawp[0, :n] + rawp[1, :n]
    dis = dis_ref[...]
    a1 = dis[:, None] * raw + (dis * dis)[:, None] * h1_ref[...] \
        + b1_ref[...][None, :]
    h = jnp.maximum(a1, 0.0)
    h_ref[...] = h
    hs2_ref[...] = h * dis[:, None]


def _head_body(rawp_ref, h_ref, dis_ref, wmu_ref, bmu_ref, wvar_ref,
               bvar_ref, eps_ref, zm_ref, zv_ref, z_ref):
    n = h_ref.shape[0]
    rawp = rawp_ref[...]
    raw = rawp[0, :n] + rawp[1, :n]
    dis = dis_ref[...]
    a2 = dis[:, None] * raw + (dis * dis)[:, None] * h_ref[...]
    zm = jnp.dot(a2, wmu_ref[...], preferred_element_type=jnp.float32) \
        + bmu_ref[...][None, :]
    pv = jnp.dot(a2, wvar_ref[...], preferred_element_type=jnp.float32) \
        + bvar_ref[...][None, :]
    zv = jnp.maximum(pv, 0.0) + jnp.log(1.0 + jnp.exp(-jnp.abs(pv)))
    zm_ref[...] = zm
    zv_ref[...] = zv
    z_ref[...] = zm + zv * eps_ref[...]


def kernel(x, edge_index, W1, b1, Wmu, bmu, Wvar, bvar):
    n, d = x.shape
    e = edge_index.shape[1]
    h = W1.shape[1]
    z = Wmu.shape[1]
    nb = e // (_NW * _SB)

    src3 = edge_index[0].reshape(_NW, nb, _SB)
    dst3 = edge_index[1].reshape(_NW, nb, _SB)

    f32 = jnp.float32
    h1 = pl.pallas_call(
        _mm_body, out_shape=jax.ShapeDtypeStruct((n, h), f32))(x, W1)

    degp = _sc_degree(dst3, nb)

    dis, hs1 = pl.pallas_call(
        _scale_body,
        out_shape=(jax.ShapeDtypeStruct((n,), f32),
                   jax.ShapeDtypeStruct((n, h), f32)))(degp, h1)

    raw1p = _sc_aggregate(hs1, src3, dst3, nb)

    hh, hs2 = pl.pallas_call(
        _hidden_body,
        out_shape=(jax.ShapeDtypeStruct((n, h), f32),
                   jax.ShapeDtypeStruct((n, h), f32)))(raw1p, h1, dis, b1)

    raw2p = _sc_aggregate(hs2, src3, dst3, nb)

    eps = jax.random.normal(jax.random.key(42), (n, z), f32)
    zm, zv, zz = pl.pallas_call(
        _head_body,
        out_shape=(jax.ShapeDtypeStruct((n, z), f32),
                   jax.ShapeDtypeStruct((n, z), f32),
                   jax.ShapeDtypeStruct((n, z), f32)))(
        raw2p, hh, dis, Wmu, bmu, Wvar, bvar, eps)
    return (zm, zv, zz)
